# baseline (speedup 1.0000x reference)
import functools

import jax
import jax.numpy as jnp
from jax import lax
from jax.experimental import pallas as pl
from jax.experimental.pallas import tpu as pltpu
from jax.experimental.pallas import tpu_sc as plsc

VOCAB = 1000000
FEATURES = 64
BATCH = 16384

NC = 2
NS = 16
NW = NC * NS
PAIRS_PER_W = BATCH // NW
IDX_CHUNK = 128
N_CHUNKS = PAIRS_PER_W // IDX_CHUNK
PACKED = FEATURES // 2


def _sc_kernel(xu_hbm, xv_hbm, w2_hbm, out_hbm, uidx_v, vidx_v,
               ru_v, rv_v, rows_u, rows_v, dots_v, sem):
    c = lax.axis_index("c")
    s = lax.axis_index("s")
    wid = s * NC + c

    pltpu.sync_copy(xu_hbm.at[wid], uidx_v)
    pltpu.sync_copy(xv_hbm.at[wid], vidx_v)

    for j in range(N_CHUNKS):
        for k in range(IDX_CHUNK // 16):
            col = pl.ds(k * 16, 16)
            ru_v[j, col] = lax.shift_right_logical(uidx_v[j, col], 2)
            rv_v[j, col] = lax.shift_right_logical(vidx_v[j, col], 2)

    lane = lax.iota(jnp.int32, 16)

    for h in range(N_CHUNKS):
        copies = [
            pltpu.async_copy(w2_hbm.at[ru_v.at[h]], rows_u, sem),
            pltpu.async_copy(w2_hbm.at[rv_v.at[h]], rows_v, sem),
        ]
        for cp in copies:
            cp.wait()

        def group_body(g, _):
            col = pl.ds(g * 16, 16)
            ou = lax.shift_left(jnp.bitwise_and(uidx_v[h, col], 3), 5)
            ov = lax.shift_left(jnp.bitwise_and(vidx_v[h, col], 3), 5)
            rowids = g * 16 + lane
            acc = jnp.zeros((16,), jnp.float32)
            for k in range(PACKED):
                rot = jnp.bitwise_and(lane + k, PACKED - 1)
                pu = plsc.load_gather(rows_u, [rowids, ou + rot])
                pv = plsc.load_gather(rows_v, [rowids, ov + rot])
                u0, u1 = plsc.unpack(plsc.bitcast(pu, jnp.bfloat16),
                                     format=plsc.PackFormat.INTERLEAVED)
                v0, v1 = plsc.unpack(plsc.bitcast(pv, jnp.bfloat16),
                                     format=plsc.PackFormat.INTERLEAVED)
                acc = acc + u0 * v0 + u1 * v1
            dots_v[pl.ds(h * IDX_CHUNK + g * 16, 16)] = (
                1.0 / (1.0 + jnp.exp(-acc)))
            return 0

        lax.fori_loop(0, IDX_CHUNK // 16, group_body, 0)

    pltpu.sync_copy(dots_v, out_hbm.at[pl.ds(wid * PAIRS_PER_W, PAIRS_PER_W)])


@jax.jit
def kernel(x, W_g):
    xt = x.T
    xu = xt[0].reshape(NW, N_CHUNKS, IDX_CHUNK)
    xv = xt[1].reshape(NW, N_CHUNKS, IDX_CHUNK)
    wu = lax.bitcast_convert_type(W_g.astype(jnp.bfloat16), jnp.uint16)
    wu = wu.reshape(VOCAB // 4, 4 * FEATURES)
    lo = wu[:, 0::2].astype(jnp.uint32)
    hi = wu[:, 1::2].astype(jnp.uint32)
    w2 = lax.bitcast_convert_type(lo | (hi << 16), jnp.int32)
    mesh = plsc.VectorSubcoreMesh(core_axis_name="c", subcore_axis_name="s")
    run = functools.partial(
        pl.kernel,
        mesh=mesh,
        out_type=jax.ShapeDtypeStruct((BATCH,), jnp.float32),
        scratch_types=[
            pltpu.VMEM((N_CHUNKS, IDX_CHUNK), jnp.int32),
            pltpu.VMEM((N_CHUNKS, IDX_CHUNK), jnp.int32),
            pltpu.VMEM((N_CHUNKS, IDX_CHUNK), jnp.int32),
            pltpu.VMEM((N_CHUNKS, IDX_CHUNK), jnp.int32),
            pltpu.VMEM((IDX_CHUNK, 2 * FEATURES), jnp.int32),
            pltpu.VMEM((IDX_CHUNK, 2 * FEATURES), jnp.int32),
            pltpu.VMEM((PAIRS_PER_W,), jnp.float32),
            pltpu.SemaphoreType.DMA,
        ],
        compiler_params=pltpu.CompilerParams(needs_layout_passes=False),
    )(_sc_kernel)
    out = run(xu, xv, w2)
    return out.reshape(BATCH, 1)


# speedup vs baseline: 16.2563x; 16.2563x over previous
import functools

import jax
import jax.numpy as jnp
from jax import lax
from jax.experimental import pallas as pl
from jax.experimental.pallas import tpu as pltpu
from jax.experimental.pallas import tpu_sc as plsc

VOCAB = 1000000
FEATURES = 64
BATCH = 16384

NC = 2
NS = 16
NW = NC * NS
PAIRS_PER_W = BATCH // NW
IDX_CHUNK = 128
N_CHUNKS = PAIRS_PER_W // IDX_CHUNK

def _sc_kernel(xu_hbm, xv_hbm, wp_hbm, out_hbm, uidx_v, vidx_v,
               rows_u, rows_v, dots_v, sem):
    c = lax.axis_index("c")
    s = lax.axis_index("s")
    wid = s * NC + c

    pltpu.sync_copy(xu_hbm.at[wid], uidx_v)
    pltpu.sync_copy(xv_hbm.at[wid], vidx_v)

    lane = lax.iota(jnp.int32, 16)

    for h in range(N_CHUNKS):
        copies = [
            pltpu.async_copy(wp_hbm.at[uidx_v.at[h]], rows_u, sem),
            pltpu.async_copy(wp_hbm.at[vidx_v.at[h]], rows_v, sem),
        ]
        for cp in copies:
            cp.wait()

        def group_body(g, _):
            rowids = g * 16 + lane
            rot0 = jnp.bitwise_and(lane, FEATURES - 1)
            acc = (plsc.load_gather(rows_u, [rowids, rot0])
                   * plsc.load_gather(rows_v, [rowids, rot0]))
            for f in range(1, FEATURES):
                rot = jnp.bitwise_and(lane + f, FEATURES - 1)
                acc = acc + (plsc.load_gather(rows_u, [rowids, rot])
                             * plsc.load_gather(rows_v, [rowids, rot]))
            dots_v[pl.ds(h * IDX_CHUNK + g * 16, 16)] = (
                1.0 / (1.0 + jnp.exp(-acc)))
            return 0

        lax.fori_loop(0, IDX_CHUNK // 16, group_body, 0)

    pltpu.sync_copy(dots_v, out_hbm.at[pl.ds(wid * PAIRS_PER_W, PAIRS_PER_W)])


@jax.jit
def kernel(x, W_g):
    xt = x.T
    xu = xt[0].reshape(NW, N_CHUNKS, IDX_CHUNK)
    xv = xt[1].reshape(NW, N_CHUNKS, IDX_CHUNK)
    wp = jnp.pad(W_g, ((0, 0), (0, FEATURES)))
    mesh = plsc.VectorSubcoreMesh(core_axis_name="c", subcore_axis_name="s")
    run = functools.partial(
        pl.kernel,
        mesh=mesh,
        out_type=jax.ShapeDtypeStruct((BATCH,), jnp.float32),
        scratch_types=[
            pltpu.VMEM((N_CHUNKS, IDX_CHUNK), jnp.int32),
            pltpu.VMEM((N_CHUNKS, IDX_CHUNK), jnp.int32),
            pltpu.VMEM((IDX_CHUNK, 2 * FEATURES), jnp.float32),
            pltpu.VMEM((IDX_CHUNK, 2 * FEATURES), jnp.float32),
            pltpu.VMEM((PAIRS_PER_W,), jnp.float32),
            pltpu.SemaphoreType.DMA,
        ],
        compiler_params=pltpu.CompilerParams(needs_layout_passes=False),
    )(_sc_kernel)
    out = run(xu, xv, wp)
    return out.reshape(BATCH, 1)


# speedup vs baseline: 16.2626x; 1.0004x over previous
import functools

import jax
import jax.numpy as jnp
from jax import lax
from jax.experimental import pallas as pl
from jax.experimental.pallas import tpu as pltpu
from jax.experimental.pallas import tpu_sc as plsc

VOCAB = 1000000
FEATURES = 64
BATCH = 16384

NC = 2
NS = 16
NW = NC * NS
PAIRS_PER_W = BATCH // NW
IDX_CHUNK = 128
N_CHUNKS = PAIRS_PER_W // IDX_CHUNK

def _sc_kernel(xu_hbm, xv_hbm, wp_hbm, out_hbm, uidx_v, vidx_v,
               rows_u, rows_v, dots_v, sem):
    c = lax.axis_index("c")
    s = lax.axis_index("s")
    wid = s * NC + c

    pltpu.sync_copy(xu_hbm.at[wid], uidx_v)
    pltpu.sync_copy(xv_hbm.at[wid], vidx_v)

    lane = lax.iota(jnp.int32, 16)

    for h in range(N_CHUNKS):
        copies = [
            pltpu.async_copy(wp_hbm.at[uidx_v.at[h]], rows_u, sem),
            pltpu.async_copy(wp_hbm.at[vidx_v.at[h]], rows_v, sem),
        ]
        for cp in copies:
            cp.wait()

        def group_body(g, _):
            rowids = g * 16 + lane
            rot0 = jnp.bitwise_and(lane, FEATURES - 1)
            acc = (plsc.load_gather(rows_u, [rowids, rot0])
                   * plsc.load_gather(rows_v, [rowids, rot0]))
            for f in range(1, FEATURES):
                rot = jnp.bitwise_and(lane + f, FEATURES - 1)
                acc = acc + (plsc.load_gather(rows_u, [rowids, rot])
                             * plsc.load_gather(rows_v, [rowids, rot]))
            dots_v[pl.ds(h * IDX_CHUNK + g * 16, 16)] = (
                1.0 / (1.0 + jnp.exp(-acc)))
            return 0

        lax.fori_loop(0, IDX_CHUNK // 16, group_body, 0)

    pltpu.sync_copy(dots_v, out_hbm.at[pl.ds(wid * PAIRS_PER_W, PAIRS_PER_W)])


@jax.jit
def kernel(x, W_g):
    xt = x.T
    xu = xt[0].reshape(NW, N_CHUNKS, IDX_CHUNK)
    xv = xt[1].reshape(NW, N_CHUNKS, IDX_CHUNK)
    wp = jnp.pad(W_g, ((0, 0), (0, FEATURES)))
    mesh = plsc.VectorSubcoreMesh(core_axis_name="c", subcore_axis_name="s")
    run = functools.partial(
        pl.kernel,
        mesh=mesh,
        out_type=jax.ShapeDtypeStruct((BATCH,), jnp.float32),
        scratch_types=[
            pltpu.VMEM((N_CHUNKS, IDX_CHUNK), jnp.int32),
            pltpu.VMEM((N_CHUNKS, IDX_CHUNK), jnp.int32),
            pltpu.VMEM((IDX_CHUNK, 2 * FEATURES), jnp.float32),
            pltpu.VMEM((IDX_CHUNK, 2 * FEATURES), jnp.float32),
            pltpu.VMEM((PAIRS_PER_W,), jnp.float32),
            pltpu.SemaphoreType.DMA,
        ],
        compiler_params=pltpu.CompilerParams(
            needs_layout_passes=False, use_tc_tiling_on_sc=False
        ),
    )(_sc_kernel)
    out = run(xu, xv, wp)
    return out.reshape(BATCH, 1)
